# baseline (speedup 1.0000x reference)
import functools

import jax
import jax.numpy as jnp
from jax import lax
from jax.experimental import pallas as pl
from jax.experimental.pallas import tpu as pltpu
from jax.experimental.pallas import tpu_sc as plsc

VOCAB = 100000
VOCAB_PAD = 100352
DIM = 128
EPS = 1e-5


def _sc_hist(x_flat, mask_f, n_rows):
    info = plsc.get_sparse_core_info()
    nc, ns = info.num_cores, info.num_subcores
    nw = nc * ns
    per_w = n_rows // nw

    mesh = plsc.VectorSubcoreMesh(core_axis_name="c", subcore_axis_name="s")

    @functools.partial(
        pl.kernel,
        mesh=mesh,
        compiler_params=pltpu.CompilerParams(needs_layout_passes=False),
        out_type=jax.ShapeDtypeStruct((nw, 1, VOCAB_PAD), jnp.float32),
        scratch_types=[
            pltpu.VMEM((VOCAB_PAD,), jnp.float32),
            pltpu.VMEM((per_w,), jnp.int32),
            pltpu.VMEM((per_w,), jnp.float32),
        ],
    )
    def hist_kernel(x_hbm, m_hbm, out_hbm, counts, xv, mv):
        wid = lax.axis_index("s") * nc + lax.axis_index("c")

        pltpu.sync_copy(x_hbm.at[pl.ds(wid * per_w, per_w)], xv)
        pltpu.sync_copy(m_hbm.at[pl.ds(wid * per_w, per_w)], mv)

        zero = jnp.zeros((16,), jnp.float32)

        def zbody(i, c):
            for u in range(8):
                counts[pl.ds((i * 8 + u) * 16, 16)] = zero
            return c

        lax.fori_loop(0, VOCAB_PAD // (16 * 8), zbody, 0)

        def inner(i, cc):
            for u in range(4):
                off = (i * 4 + u) * 16
                idx = xv[pl.ds(off, 16)]
                val = mv[pl.ds(off, 16)]
                plsc.addupdate_scatter(counts, [idx], val)
            return cc

        lax.fori_loop(0, per_w // (16 * 4), inner, 0)
        pltpu.sync_copy(counts, out_hbm.at[wid, 0])

    return hist_kernel(x_flat, mask_f)


_GSTREAM = 128
_CHUNK = 256
_NSLOT = 3


def _sc_gather(x_flat, table, scale, shift, n_rows):
    info = plsc.get_sparse_core_info()
    nc, ns = info.num_cores, info.num_subcores
    nw = nc * ns
    rows_per_w = n_rows // nw
    chunks_per_w = rows_per_w // _CHUNK

    mesh = plsc.VectorSubcoreMesh(core_axis_name="c", subcore_axis_name="s")

    @functools.partial(
        pl.kernel,
        mesh=mesh,
        compiler_params=pltpu.CompilerParams(needs_layout_passes=False),
        out_type=jax.ShapeDtypeStruct((n_rows, DIM), jnp.float32),
        scratch_types=[
            pltpu.VMEM((rows_per_w,), jnp.int32),
            pltpu.VMEM((DIM,), jnp.float32),
            pltpu.VMEM((DIM,), jnp.float32),
            pltpu.VMEM((_CHUNK, DIM), jnp.float32),
            pltpu.VMEM((_CHUNK, DIM), jnp.float32),
            pltpu.VMEM((_CHUNK, DIM), jnp.float32),
            pltpu.SemaphoreType.DMA,
            pltpu.SemaphoreType.DMA,
            pltpu.SemaphoreType.DMA,
            pltpu.SemaphoreType.DMA,
            pltpu.SemaphoreType.DMA,
            pltpu.SemaphoreType.DMA,
        ],
    )
    def gather_kernel(x_hbm, table_hbm, scale_hbm, shift_hbm, out_hbm, idx_v,
                      scale_v, shift_v,
                      b0, b1, b2, sg0, sg1, sg2, ss0, ss1, ss2):
        wid = lax.axis_index("s") * nc + lax.axis_index("c")
        base_out_row = wid * rows_per_w
        pltpu.sync_copy(x_hbm.at[pl.ds(wid * rows_per_w, rows_per_w)], idx_v)
        pltpu.sync_copy(scale_hbm.at[0], scale_v)
        pltpu.sync_copy(shift_hbm.at[0], shift_v)

        svecs = [scale_v[pl.ds(g * 16, 16)] for g in range(DIM // 16)]
        bvecs = [shift_v[pl.ds(g * 16, 16)] for g in range(DIM // 16)]

        bufs = (b0, b1, b2)
        sgs = (sg0, sg1, sg2)
        sss = (ss0, ss1, ss2)

        def start_gathers(j, buf, sem):
            pltpu.async_copy(
                table_hbm.at[idx_v.at[pl.ds(j * _CHUNK, _GSTREAM)]],
                buf.at[pl.ds(0, _GSTREAM)], sem)
            pltpu.async_copy(
                table_hbm.at[idx_v.at[pl.ds(j * _CHUNK + _GSTREAM, _GSTREAM)]],
                buf.at[pl.ds(_GSTREAM, _GSTREAM)], sem)

        def wait_gathers(j, buf, sem):
            for h in range(2):
                pltpu.make_async_copy(
                    table_hbm.at[idx_v.at[pl.ds(j * _CHUNK + h * _GSTREAM,
                                                _GSTREAM)]],
                    buf.at[pl.ds(h * _GSTREAM, _GSTREAM)], sem).wait()

        def affine(buf):
            def rbody(r, cc):
                for g in range(DIM // 16):
                    v = buf[r, pl.ds(g * 16, 16)]
                    buf[r, pl.ds(g * 16, 16)] = v * svecs[g] + bvecs[g]
                return cc

            lax.fori_loop(0, _CHUNK, rbody, 0)

        for k in range(_NSLOT - 1):
            start_gathers(k, bufs[k], sgs[k])

        def body(j, carry):
            for k in range(_NSLOT):

                @pl.when(j % _NSLOT == k)
                def _():
                    m = (k + _NSLOT - 1) % _NSLOT
                    wait_gathers(j, bufs[k], sgs[k])
                    @pl.when(jnp.logical_and(j >= 1, j + 2 < chunks_per_w))
                    def _():
                        pltpu.make_async_copy(
                            bufs[m],
                            out_hbm.at[pl.ds(base_out_row, _CHUNK)],
                            sss[m],
                        ).wait()

                    @pl.when(j + 2 < chunks_per_w)
                    def _():
                        start_gathers(j + 2, bufs[m], sgs[m])

                    affine(bufs[k])
                    pltpu.async_copy(
                        bufs[k],
                        out_hbm.at[pl.ds(base_out_row + j * _CHUNK, _CHUNK)],
                        sss[k],
                    )

            return carry

        lax.fori_loop(0, chunks_per_w, body, 0)

        for k in range(_NSLOT):
            pltpu.make_async_copy(
                bufs[k], out_hbm.at[pl.ds(base_out_row, _CHUNK)], sss[k]
            ).wait()

    return gather_kernel(x_flat, table, scale, shift)


_VBLK = 2048
_NSTEPS = VOCAB_PAD // _VBLK


def _stats_body(h_ref, tbl_ref, tail_ref, gamma_ref, beta_ref,
                scale_ref, shift_ref, acc_ref):
    i = pl.program_id(0)

    @pl.when(i == 0)
    def _():
        acc_ref[...] = jnp.zeros_like(acc_ref)

    cblk = jnp.sum(h_ref[:, 0, :], axis=0, keepdims=True)

    def accum(tblk):
        s = lax.dot_general(cblk, tblk, (((1,), (0,)), ((), ())),
                            preferred_element_type=jnp.float32)
        sq = lax.dot_general(cblk, tblk * tblk, (((1,), (0,)), ((), ())),
                             preferred_element_type=jnp.float32)
        acc_ref[0:1, :] += s
        acc_ref[1:2, :] += sq
        acc_ref[2:3, :] += jnp.full((1, DIM), jnp.sum(cblk), jnp.float32)

    @pl.when(i < _NSTEPS - 1)
    def _():
        accum(tbl_ref[...])

    @pl.when(i == _NSTEPS - 1)
    def _():
        accum(tail_ref[...])

        cnt = jnp.maximum(acc_ref[2:3, :], 1.0)
        mean = acc_ref[0:1, :] / cnt
        var = jnp.maximum(acc_ref[1:2, :] / cnt - mean * mean, 0.0)
        scale = gamma_ref[...] / jnp.sqrt(var + EPS)
        scale_ref[...] = scale
        shift_ref[...] = beta_ref[...] - mean * scale


def _tc_stats(hists, table, gamma, beta):
    tail = jnp.concatenate(
        [table[(_NSTEPS - 1) * _VBLK:],
         jnp.zeros((VOCAB_PAD - VOCAB, DIM), jnp.float32)], axis=0)
    return pl.pallas_call(
        _stats_body,
        grid=(_NSTEPS,),
        in_specs=[
            pl.BlockSpec((32, 1, _VBLK), lambda i: (0, 0, i)),
            pl.BlockSpec((_VBLK, DIM),
                         lambda i: (jnp.minimum(i, _NSTEPS - 2), 0)),
            pl.BlockSpec((_VBLK, DIM), lambda i: (0, 0)),
            pl.BlockSpec((1, DIM), lambda i: (0, 0)),
            pl.BlockSpec((1, DIM), lambda i: (0, 0)),
        ],
        out_specs=[
            pl.BlockSpec((1, DIM), lambda i: (0, 0)),
            pl.BlockSpec((1, DIM), lambda i: (0, 0)),
        ],
        out_shape=[
            jax.ShapeDtypeStruct((1, DIM), jnp.float32),
            jax.ShapeDtypeStruct((1, DIM), jnp.float32),
        ],
        scratch_shapes=[pltpu.VMEM((3, DIM), jnp.float32)],
    )(hists, table, tail, gamma.reshape(1, DIM), beta.reshape(1, DIM))


def kernel(x, mask, table, gamma, beta):
    b, l = x.shape
    n_rows = b * l
    x_flat = x.reshape(n_rows).astype(jnp.int32)
    mask_f = mask.reshape(n_rows).astype(jnp.float32)
    hists = _sc_hist(x_flat, mask_f, n_rows)
    scale, shift = _tc_stats(hists, table, gamma, beta)
    out = _sc_gather(x_flat, table, scale, shift, n_rows)
    return out.reshape(b, l, DIM)


# speedup vs baseline: 1.1363x; 1.1363x over previous
import functools

import jax
import jax.numpy as jnp
from jax import lax
from jax.experimental import pallas as pl
from jax.experimental.pallas import tpu as pltpu
from jax.experimental.pallas import tpu_sc as plsc

VOCAB = 100000
VOCAB_PAD = 100352
DIM = 128
EPS = 1e-5


def _sc_hist(x_flat, mask_f, n_rows):
    info = plsc.get_sparse_core_info()
    nc, ns = info.num_cores, info.num_subcores
    nw = nc * ns
    per_w = n_rows // nw

    mesh = plsc.VectorSubcoreMesh(core_axis_name="c", subcore_axis_name="s")

    @functools.partial(
        pl.kernel,
        mesh=mesh,
        compiler_params=pltpu.CompilerParams(needs_layout_passes=False),
        out_type=jax.ShapeDtypeStruct((nw, 1, VOCAB_PAD), jnp.float32),
        scratch_types=[
            pltpu.VMEM((VOCAB_PAD,), jnp.float32),
            pltpu.VMEM((per_w,), jnp.int32),
            pltpu.VMEM((per_w,), jnp.float32),
        ],
    )
    def hist_kernel(x_hbm, m_hbm, out_hbm, counts, xv, mv):
        wid = lax.axis_index("s") * nc + lax.axis_index("c")

        pltpu.sync_copy(x_hbm.at[pl.ds(wid * per_w, per_w)], xv)
        pltpu.sync_copy(m_hbm.at[pl.ds(wid * per_w, per_w)], mv)

        zero = jnp.zeros((16,), jnp.float32)

        def zbody(i, c):
            for u in range(8):
                counts[pl.ds((i * 8 + u) * 16, 16)] = zero
            return c

        lax.fori_loop(0, VOCAB_PAD // (16 * 8), zbody, 0)

        def inner(i, cc):
            for u in range(4):
                off = (i * 4 + u) * 16
                idx = xv[pl.ds(off, 16)]
                val = mv[pl.ds(off, 16)]
                plsc.addupdate_scatter(counts, [idx], val)
            return cc

        lax.fori_loop(0, per_w // (16 * 4), inner, 0)
        pltpu.sync_copy(counts, out_hbm.at[wid, 0])

    return hist_kernel(x_flat, mask_f)


_GSTREAM = 128
_CHUNK = 256
_NSLOT = 3


def _sc_gather(x_flat, table, scale, shift, n_rows):
    info = plsc.get_sparse_core_info()
    nc, ns = info.num_cores, info.num_subcores
    nw = nc * ns
    rows_per_w = n_rows // nw
    chunks_per_w = rows_per_w // _CHUNK

    mesh = plsc.VectorSubcoreMesh(core_axis_name="c", subcore_axis_name="s")

    @functools.partial(
        pl.kernel,
        mesh=mesh,
        compiler_params=pltpu.CompilerParams(needs_layout_passes=False),
        out_type=jax.ShapeDtypeStruct((n_rows, DIM), jnp.float32),
        scratch_types=[
            pltpu.VMEM((rows_per_w,), jnp.int32),
            pltpu.VMEM((DIM,), jnp.float32),
            pltpu.VMEM((DIM,), jnp.float32),
            pltpu.VMEM((_CHUNK, DIM), jnp.float32),
            pltpu.VMEM((_CHUNK, DIM), jnp.float32),
            pltpu.VMEM((_CHUNK, DIM), jnp.float32),
            pltpu.SemaphoreType.DMA,
            pltpu.SemaphoreType.DMA,
            pltpu.SemaphoreType.DMA,
            pltpu.SemaphoreType.DMA,
            pltpu.SemaphoreType.DMA,
            pltpu.SemaphoreType.DMA,
        ],
    )
    def gather_kernel(x_hbm, table_hbm, scale_hbm, shift_hbm, out_hbm, idx_v,
                      scale_v, shift_v,
                      b0, b1, b2, sg0, sg1, sg2, ss0, ss1, ss2):
        wid = lax.axis_index("s") * nc + lax.axis_index("c")
        base_out_row = wid * rows_per_w
        pltpu.sync_copy(x_hbm.at[pl.ds(wid * rows_per_w, rows_per_w)], idx_v)
        pltpu.sync_copy(scale_hbm.at[0], scale_v)
        pltpu.sync_copy(shift_hbm.at[0], shift_v)

        svecs = [scale_v[pl.ds(g * 16, 16)] for g in range(DIM // 16)]
        bvecs = [shift_v[pl.ds(g * 16, 16)] for g in range(DIM // 16)]

        bufs = (b0, b1, b2)
        sgs = (sg0, sg1, sg2)
        sss = (ss0, ss1, ss2)

        def start_gathers(j, buf, sem):
            pltpu.async_copy(
                table_hbm.at[idx_v.at[pl.ds(j * _CHUNK, _GSTREAM)]],
                buf.at[pl.ds(0, _GSTREAM)], sem)
            pltpu.async_copy(
                table_hbm.at[idx_v.at[pl.ds(j * _CHUNK + _GSTREAM, _GSTREAM)]],
                buf.at[pl.ds(_GSTREAM, _GSTREAM)], sem)

        def wait_gathers(j, buf, sem):
            for h in range(2):
                pltpu.make_async_copy(
                    table_hbm.at[idx_v.at[pl.ds(j * _CHUNK + h * _GSTREAM,
                                                _GSTREAM)]],
                    buf.at[pl.ds(h * _GSTREAM, _GSTREAM)], sem).wait()

        def affine(buf):
            def rbody(r, cc):
                for g in range(DIM // 16):
                    v = buf[r, pl.ds(g * 16, 16)]
                    buf[r, pl.ds(g * 16, 16)] = v * svecs[g] + bvecs[g]
                return cc

            lax.fori_loop(0, _CHUNK, rbody, 0)

        for k in range(_NSLOT - 1):
            start_gathers(k, bufs[k], sgs[k])

        def body(j, carry):
            for k in range(_NSLOT):

                @pl.when(j % _NSLOT == k)
                def _():
                    m = (k + _NSLOT - 1) % _NSLOT
                    wait_gathers(j, bufs[k], sgs[k])
                    @pl.when(jnp.logical_and(j >= 1, j + 2 < chunks_per_w))
                    def _():
                        pltpu.make_async_copy(
                            bufs[m],
                            out_hbm.at[pl.ds(base_out_row, _CHUNK)],
                            sss[m],
                        ).wait()

                    @pl.when(j + 2 < chunks_per_w)
                    def _():
                        start_gathers(j + 2, bufs[m], sgs[m])

                    affine(bufs[k])
                    pltpu.async_copy(
                        bufs[k],
                        out_hbm.at[pl.ds(base_out_row + j * _CHUNK, _CHUNK)],
                        sss[k],
                    )

            return carry

        lax.fori_loop(0, chunks_per_w, body, 0)

        for k in range(_NSLOT):
            pltpu.make_async_copy(
                bufs[k], out_hbm.at[pl.ds(base_out_row, _CHUNK)], sss[k]
            ).wait()

    return gather_kernel(x_flat, table, scale, shift)


_VBLK = 7168
_NSTEPS = VOCAB_PAD // _VBLK


def _stats_body(h_ref, tbl_ref, tail_ref, gamma_ref, beta_ref,
                scale_ref, shift_ref, acc_ref):
    i = pl.program_id(0)

    @pl.when(i == 0)
    def _():
        acc_ref[...] = jnp.zeros_like(acc_ref)

    cblk = jnp.sum(h_ref[:, 0, :], axis=0, keepdims=True)

    def accum(tblk):
        s = lax.dot_general(cblk, tblk, (((1,), (0,)), ((), ())),
                            preferred_element_type=jnp.float32)
        sq = lax.dot_general(cblk, tblk * tblk, (((1,), (0,)), ((), ())),
                             preferred_element_type=jnp.float32)
        acc_ref[0:1, :] += s
        acc_ref[1:2, :] += sq
        acc_ref[2:3, :] += jnp.full((1, DIM), jnp.sum(cblk), jnp.float32)

    @pl.when(i < _NSTEPS - 1)
    def _():
        accum(tbl_ref[...])

    @pl.when(i == _NSTEPS - 1)
    def _():
        accum(tail_ref[...])

        cnt = jnp.maximum(acc_ref[2:3, :], 1.0)
        mean = acc_ref[0:1, :] / cnt
        var = jnp.maximum(acc_ref[1:2, :] / cnt - mean * mean, 0.0)
        scale = gamma_ref[...] / jnp.sqrt(var + EPS)
        scale_ref[...] = scale
        shift_ref[...] = beta_ref[...] - mean * scale


def _tc_stats(hists, table, gamma, beta):
    tail = jnp.concatenate(
        [table[(_NSTEPS - 1) * _VBLK:],
         jnp.zeros((VOCAB_PAD - VOCAB, DIM), jnp.float32)], axis=0)
    return pl.pallas_call(
        _stats_body,
        grid=(_NSTEPS,),
        in_specs=[
            pl.BlockSpec((32, 1, _VBLK), lambda i: (0, 0, i)),
            pl.BlockSpec((_VBLK, DIM),
                         lambda i: (jnp.minimum(i, _NSTEPS - 2), 0)),
            pl.BlockSpec((_VBLK, DIM), lambda i: (0, 0)),
            pl.BlockSpec((1, DIM), lambda i: (0, 0)),
            pl.BlockSpec((1, DIM), lambda i: (0, 0)),
        ],
        out_specs=[
            pl.BlockSpec((1, DIM), lambda i: (0, 0)),
            pl.BlockSpec((1, DIM), lambda i: (0, 0)),
        ],
        out_shape=[
            jax.ShapeDtypeStruct((1, DIM), jnp.float32),
            jax.ShapeDtypeStruct((1, DIM), jnp.float32),
        ],
        scratch_shapes=[pltpu.VMEM((3, DIM), jnp.float32)],
    )(hists, table, tail, gamma.reshape(1, DIM), beta.reshape(1, DIM))


def kernel(x, mask, table, gamma, beta):
    b, l = x.shape
    n_rows = b * l
    x_flat = x.reshape(n_rows).astype(jnp.int32)
    mask_f = mask.reshape(n_rows).astype(jnp.float32)
    hists = _sc_hist(x_flat, mask_f, n_rows)
    scale, shift = _tc_stats(hists, table, gamma, beta)
    out = _sc_gather(x_flat, table, scale, shift, n_rows)
    return out.reshape(b, l, DIM)


# speedup vs baseline: 1.1598x; 1.0207x over previous
import functools

import jax
import jax.numpy as jnp
from jax import lax
from jax.experimental import pallas as pl
from jax.experimental.pallas import tpu as pltpu
from jax.experimental.pallas import tpu_sc as plsc

VOCAB = 100000
VOCAB_PAD = 100352
DIM = 128
EPS = 1e-5


def _sc_hist(x_flat, mask_f, n_rows):
    info = plsc.get_sparse_core_info()
    nc, ns = info.num_cores, info.num_subcores
    nw = nc * ns
    per_w = n_rows // nw

    mesh = plsc.VectorSubcoreMesh(core_axis_name="c", subcore_axis_name="s")

    @functools.partial(
        pl.kernel,
        mesh=mesh,
        compiler_params=pltpu.CompilerParams(needs_layout_passes=False),
        out_type=jax.ShapeDtypeStruct((nw, 1, VOCAB_PAD), jnp.float32),
        scratch_types=[
            pltpu.VMEM((VOCAB_PAD,), jnp.float32),
            pltpu.VMEM((per_w,), jnp.int32),
            pltpu.VMEM((per_w,), jnp.float32),
        ],
    )
    def hist_kernel(x_hbm, m_hbm, out_hbm, counts, xv, mv):
        wid = lax.axis_index("s") * nc + lax.axis_index("c")

        pltpu.sync_copy(x_hbm.at[pl.ds(wid * per_w, per_w)], xv)
        pltpu.sync_copy(m_hbm.at[pl.ds(wid * per_w, per_w)], mv)

        zero = jnp.zeros((16,), jnp.float32)

        def zbody(i, c):
            for u in range(8):
                counts[pl.ds((i * 8 + u) * 16, 16)] = zero
            return c

        lax.fori_loop(0, VOCAB_PAD // (16 * 8), zbody, 0)

        def inner(i, cc):
            for u in range(4):
                off = (i * 4 + u) * 16
                idx = xv[pl.ds(off, 16)]
                val = mv[pl.ds(off, 16)]
                plsc.addupdate_scatter(counts, [idx], val)
            return cc

        lax.fori_loop(0, per_w // (16 * 4), inner, 0)
        pltpu.sync_copy(counts, out_hbm.at[wid, 0])

    return hist_kernel(x_flat, mask_f)


_GSTREAM = 128
_CHUNK = 256
_NSLOT = 3


def _sc_gather(x_flat, table, scale, shift, n_rows):
    info = plsc.get_sparse_core_info()
    nc, ns = info.num_cores, info.num_subcores
    nw = nc * ns
    rows_per_w = n_rows // nw
    chunks_per_w = rows_per_w // _CHUNK

    mesh = plsc.VectorSubcoreMesh(core_axis_name="c", subcore_axis_name="s")

    @functools.partial(
        pl.kernel,
        mesh=mesh,
        compiler_params=pltpu.CompilerParams(needs_layout_passes=False),
        out_type=jax.ShapeDtypeStruct((n_rows, DIM), jnp.float32),
        scratch_types=[
            pltpu.VMEM((rows_per_w,), jnp.int32),
            pltpu.VMEM((DIM,), jnp.float32),
            pltpu.VMEM((DIM,), jnp.float32),
            pltpu.VMEM((_CHUNK, DIM), jnp.float32),
            pltpu.VMEM((_CHUNK, DIM), jnp.float32),
            pltpu.VMEM((_CHUNK, DIM), jnp.float32),
            pltpu.SemaphoreType.DMA,
            pltpu.SemaphoreType.DMA,
            pltpu.SemaphoreType.DMA,
            pltpu.SemaphoreType.DMA,
            pltpu.SemaphoreType.DMA,
            pltpu.SemaphoreType.DMA,
        ],
    )
    def gather_kernel(x_hbm, table_hbm, scale_hbm, shift_hbm, out_hbm, idx_v,
                      scale_v, shift_v,
                      b0, b1, b2, sg0, sg1, sg2, ss0, ss1, ss2):
        wid = lax.axis_index("s") * nc + lax.axis_index("c")
        base_out_row = wid * rows_per_w
        pltpu.sync_copy(x_hbm.at[pl.ds(wid * rows_per_w, rows_per_w)], idx_v)
        pltpu.sync_copy(scale_hbm.at[0], scale_v)
        pltpu.sync_copy(shift_hbm.at[0], shift_v)

        svecs = [scale_v[pl.ds(g * 16, 16)] for g in range(DIM // 16)]
        bvecs = [shift_v[pl.ds(g * 16, 16)] for g in range(DIM // 16)]

        bufs = (b0, b1, b2)
        sgs = (sg0, sg1, sg2)
        sss = (ss0, ss1, ss2)

        def start_gathers(j, buf, sem):
            pltpu.async_copy(
                table_hbm.at[idx_v.at[pl.ds(j * _CHUNK, _GSTREAM)]],
                buf.at[pl.ds(0, _GSTREAM)], sem)
            pltpu.async_copy(
                table_hbm.at[idx_v.at[pl.ds(j * _CHUNK + _GSTREAM, _GSTREAM)]],
                buf.at[pl.ds(_GSTREAM, _GSTREAM)], sem)

        def wait_gathers(j, buf, sem):
            for h in range(2):
                pltpu.make_async_copy(
                    table_hbm.at[idx_v.at[pl.ds(j * _CHUNK + h * _GSTREAM,
                                                _GSTREAM)]],
                    buf.at[pl.ds(h * _GSTREAM, _GSTREAM)], sem).wait()

        def affine(buf):
            def rbody(r, cc):
                for g in range(DIM // 16):
                    v = buf[r, pl.ds(g * 16, 16)]
                    buf[r, pl.ds(g * 16, 16)] = v * svecs[g] + bvecs[g]
                return cc

            lax.fori_loop(0, _CHUNK, rbody, 0)

        for k in range(_NSLOT - 1):
            start_gathers(k, bufs[k], sgs[k])

        def body(j, carry):
            for k in range(_NSLOT):

                @pl.when(j % _NSLOT == k)
                def _():
                    m = (k + _NSLOT - 1) % _NSLOT
                    wait_gathers(j, bufs[k], sgs[k])
                    @pl.when(jnp.logical_and(j >= 1, j + 2 < chunks_per_w))
                    def _():
                        pltpu.make_async_copy(
                            bufs[m],
                            out_hbm.at[pl.ds(base_out_row, _CHUNK)],
                            sss[m],
                        ).wait()

                    @pl.when(j + 2 < chunks_per_w)
                    def _():
                        start_gathers(j + 2, bufs[m], sgs[m])

                    affine(bufs[k])
                    pltpu.async_copy(
                        bufs[k],
                        out_hbm.at[pl.ds(base_out_row + j * _CHUNK, _CHUNK)],
                        sss[k],
                    )

            return carry

        lax.fori_loop(0, chunks_per_w, body, 0)

        for k in range(_NSLOT):
            pltpu.make_async_copy(
                bufs[k], out_hbm.at[pl.ds(base_out_row, _CHUNK)], sss[k]
            ).wait()

    return gather_kernel(x_flat, table, scale, shift)


_VBLK = 14336
_NSTEPS = VOCAB_PAD // _VBLK


def _stats_body(h_ref, tbl_ref, tail_ref, gamma_ref, beta_ref,
                scale_ref, shift_ref, acc_ref):
    i = pl.program_id(0)

    @pl.when(i == 0)
    def _():
        acc_ref[...] = jnp.zeros_like(acc_ref)

    cblk = jnp.sum(h_ref[:, 0, :], axis=0, keepdims=True)

    def accum(tblk):
        s = lax.dot_general(cblk, tblk, (((1,), (0,)), ((), ())),
                            preferred_element_type=jnp.float32)
        sq = lax.dot_general(cblk, tblk * tblk, (((1,), (0,)), ((), ())),
                             preferred_element_type=jnp.float32)
        acc_ref[0:1, :] += s
        acc_ref[1:2, :] += sq
        acc_ref[2:3, :] += jnp.full((1, DIM), jnp.sum(cblk), jnp.float32)

    @pl.when(i < _NSTEPS - 1)
    def _():
        accum(tbl_ref[...])

    @pl.when(i == _NSTEPS - 1)
    def _():
        accum(tail_ref[...])

        cnt = jnp.maximum(acc_ref[2:3, :], 1.0)
        mean = acc_ref[0:1, :] / cnt
        var = jnp.maximum(acc_ref[1:2, :] / cnt - mean * mean, 0.0)
        scale = gamma_ref[...] / jnp.sqrt(var + EPS)
        scale_ref[...] = scale
        shift_ref[...] = beta_ref[...] - mean * scale


def _tc_stats(hists, table, gamma, beta):
    tail = jnp.concatenate(
        [table[(_NSTEPS - 1) * _VBLK:],
         jnp.zeros((VOCAB_PAD - VOCAB, DIM), jnp.float32)], axis=0)
    return pl.pallas_call(
        _stats_body,
        grid=(_NSTEPS,),
        in_specs=[
            pl.BlockSpec((32, 1, _VBLK), lambda i: (0, 0, i)),
            pl.BlockSpec((_VBLK, DIM),
                         lambda i: (jnp.minimum(i, _NSTEPS - 2), 0)),
            pl.BlockSpec((_VBLK, DIM), lambda i: (0, 0)),
            pl.BlockSpec((1, DIM), lambda i: (0, 0)),
            pl.BlockSpec((1, DIM), lambda i: (0, 0)),
        ],
        out_specs=[
            pl.BlockSpec((1, DIM), lambda i: (0, 0)),
            pl.BlockSpec((1, DIM), lambda i: (0, 0)),
        ],
        out_shape=[
            jax.ShapeDtypeStruct((1, DIM), jnp.float32),
            jax.ShapeDtypeStruct((1, DIM), jnp.float32),
        ],
        scratch_shapes=[pltpu.VMEM((3, DIM), jnp.float32)],
    )(hists, table, tail, gamma.reshape(1, DIM), beta.reshape(1, DIM))


def kernel(x, mask, table, gamma, beta):
    b, l = x.shape
    n_rows = b * l
    x_flat = x.reshape(n_rows).astype(jnp.int32)
    mask_f = mask.reshape(n_rows).astype(jnp.float32)
    hists = _sc_hist(x_flat, mask_f, n_rows)
    scale, shift = _tc_stats(hists, table, gamma, beta)
    out = _sc_gather(x_flat, table, scale, shift, n_rows)
    return out.reshape(b, l, DIM)
